# baseline (speedup 1.0000x reference)
import functools

import jax
import jax.numpy as jnp
from jax import lax
from jax.experimental import pallas as pl
from jax.experimental.pallas import tpu as pltpu
from jax.experimental.pallas import tpu_sc as plsc

B = 16384
F = 26
D = 16
DIN = F * D
H0, H1, H2 = 1024, 512, 256
EPS = 1e-5
NIDX = B * F
TOTAL = 2600000

_NC, _NS = 2, 16
_NW = _NC * _NS
_PER_W = NIDX // _NW
_G = 128
_GROUPS = _PER_W // _G
_CH_G = 8
_CH = _CH_G * _G
_NCH = _GROUPS // _CH_G


def _sc_gather(xf, dims, o1f):
    @functools.partial(
        pl.kernel,
        out_type=(
            jax.ShapeDtypeStruct((D, NIDX), jnp.float32),
            jax.ShapeDtypeStruct((NIDX,), jnp.float32),
        ),
        mesh=plsc.VectorSubcoreMesh(core_axis_name="c", subcore_axis_name="s"),
        compiler_params=pltpu.CompilerParams(use_tc_tiling_on_sc=False),
        scratch_types=[
            pltpu.VMEM((_CH_G, _G), jnp.int32),
            pltpu.VMEM((D, _CH), jnp.float32),
            pltpu.VMEM((_CH,), jnp.float32),
            pltpu.SemaphoreType.DMA,
        ],
    )
    def k(idx_hbm, *refs):
        (t_hbm, o1_hbm, eT_out, o1_out, idx_v, rows_v, o1_v, sem) = (
            refs[:D], refs[D], refs[D + 1], refs[D + 2], refs[D + 3],
            refs[D + 4], refs[D + 5], refs[D + 6])
        wid = lax.axis_index("s") * _NC + lax.axis_index("c")

        def chunk(i, carry):
            row0 = wid * _GROUPS + i * _CH_G
            base = wid * _PER_W + i * _CH
            pltpu.sync_copy(idx_hbm.at[pl.ds(row0, _CH_G)], idx_v)
            for g in range(_CH_G):
                pltpu.async_copy(
                    o1_hbm.at[0].at[idx_v.at[g]], o1_v.at[pl.ds(g * _G, _G)],
                    sem)
            for d in range(D):
                for g in range(_CH_G):
                    pltpu.async_copy(
                        t_hbm[d].at[0].at[idx_v.at[g]],
                        rows_v.at[d, pl.ds(g * _G, _G)], sem)
            pltpu.make_async_copy(eT_out.at[:, pl.ds(0, _CH)], rows_v, sem).wait()
            pltpu.make_async_copy(o1_out.at[pl.ds(0, _CH)], o1_v, sem).wait()
            pltpu.sync_copy(rows_v, eT_out.at[:, pl.ds(base, _CH)])
            pltpu.sync_copy(o1_v, o1_out.at[pl.ds(base, _CH)])
            return carry

        lax.fori_loop(0, _NCH, chunk, 0)

    return k(xf, *dims, o1f)


_BM = 512
_NB = B // _BM


def _k1_body(x_ref, w0_ref, b0_ref, g1_ref, bt1_ref, s1_ref, t1_ref,
             c_acc, sum_acc):
    i = pl.program_id(0)

    @pl.when(i == 0)
    def _():
        c_acc[...] = jnp.zeros_like(c_acc)
        sum_acc[...] = jnp.zeros_like(sum_acc)

    xb = x_ref[...]
    c_acc[...] += lax.dot_general(
        xb, xb, (((0,), (0,)), ((), ())), preferred_element_type=jnp.float32)
    sum_acc[...] += jnp.sum(xb, axis=0, keepdims=True)

    @pl.when(i == _NB - 1)
    def _():
        c = c_acc[...] * (1.0 / B)
        m = sum_acc[...] * (1.0 / B)
        w0 = w0_ref[...]
        b0 = b0_ref[...]
        p = jnp.dot(c, w0, preferred_element_type=jnp.float32)
        quad = jnp.sum(w0 * p, axis=0, keepdims=True)
        mm = jnp.dot(m, w0, preferred_element_type=jnp.float32)
        mean = mm + b0
        ex2 = quad + 2.0 * b0 * mm + b0 * b0
        var = ex2 - mean * mean
        s1 = g1_ref[...] * lax.rsqrt(var + EPS)
        s1_ref[...] = s1
        t1_ref[...] = bt1_ref[...] - mean * s1


def _k1(x, w0, b0, g1, bt1):
    return pl.pallas_call(
        _k1_body,
        grid=(_NB,),
        in_specs=[
            pl.BlockSpec((_BM, DIN), lambda i: (i, 0)),
            pl.BlockSpec((DIN, H0), lambda i: (0, 0)),
            pl.BlockSpec((1, H0), lambda i: (0, 0)),
            pl.BlockSpec((1, H0), lambda i: (0, 0)),
            pl.BlockSpec((1, H0), lambda i: (0, 0)),
        ],
        out_specs=[
            pl.BlockSpec((1, H0), lambda i: (0, 0)),
            pl.BlockSpec((1, H0), lambda i: (0, 0)),
        ],
        out_shape=[
            jax.ShapeDtypeStruct((1, H0), jnp.float32),
            jax.ShapeDtypeStruct((1, H0), jnp.float32),
        ],
        scratch_shapes=[
            pltpu.VMEM((DIN, DIN), jnp.float32),
            pltpu.VMEM((1, DIN), jnp.float32),
        ],
    )(x, w0, b0, g1, bt1)


def _k2_body(x_ref, o1v_ref, w0_ref, b0_ref, s1_ref, t1_ref, w1_ref, b1_ref,
             g2_ref, bt2_ref, h1_ref, side_ref, s2_ref, t2_ref,
             sum_acc, sq_acc):
    i = pl.program_id(0)

    @pl.when(i == 0)
    def _():
        sum_acc[...] = jnp.zeros_like(sum_acc)
        sq_acc[...] = jnp.zeros_like(sq_acc)

    xb = x_ref[...]
    h0 = jnp.dot(xb, w0_ref[...], preferred_element_type=jnp.float32) + b0_ref[...]
    a1 = jnp.maximum(h0 * s1_ref[...] + t1_ref[...], 0.0)
    h1 = jnp.dot(a1, w1_ref[...], preferred_element_type=jnp.float32) + b1_ref[...]
    h1_ref[...] = h1
    sum_acc[...] += jnp.sum(h1, axis=0, keepdims=True)
    sq_acc[...] += jnp.sum(h1 * h1, axis=0, keepdims=True)

    rows = lax.broadcasted_iota(jnp.int32, (DIN, D), 0)
    cols = lax.broadcasted_iota(jnp.int32, (DIN, D), 1)
    sel = jnp.where(rows % D == cols, 1.0, 0.0).astype(jnp.float32)
    se = jnp.dot(xb, sel, preferred_element_type=jnp.float32)
    sq = jnp.dot(xb * xb, sel, preferred_element_type=jnp.float32)
    o2 = 0.5 * jnp.sum(se * se - sq, axis=1, keepdims=True)
    o1 = lax.dot_general(o1v_ref[...], jnp.ones((F, 1), jnp.float32),
                         (((1,), (0,)), ((), ())),
                         preferred_element_type=jnp.float32)
    side_ref[...] = o1 + o2

    @pl.when(i == _NB - 1)
    def _():
        mean = sum_acc[...] * (1.0 / B)
        var = sq_acc[...] * (1.0 / B) - mean * mean
        s2 = g2_ref[...] * lax.rsqrt(var + EPS)
        s2_ref[...] = s2
        t2_ref[...] = bt2_ref[...] - mean * s2


def _k2(x, o1v, w0, b0, s1, t1, w1, b1, g2, bt2):
    return pl.pallas_call(
        _k2_body,
        grid=(_NB,),
        in_specs=[
            pl.BlockSpec((_BM, DIN), lambda i: (i, 0)),
            pl.BlockSpec((_BM, F), lambda i: (i, 0)),
            pl.BlockSpec((DIN, H0), lambda i: (0, 0)),
            pl.BlockSpec((1, H0), lambda i: (0, 0)),
            pl.BlockSpec((1, H0), lambda i: (0, 0)),
            pl.BlockSpec((1, H0), lambda i: (0, 0)),
            pl.BlockSpec((H0, H1), lambda i: (0, 0)),
            pl.BlockSpec((1, H1), lambda i: (0, 0)),
            pl.BlockSpec((1, H1), lambda i: (0, 0)),
            pl.BlockSpec((1, H1), lambda i: (0, 0)),
        ],
        out_specs=[
            pl.BlockSpec((_BM, H1), lambda i: (i, 0)),
            pl.BlockSpec((_BM, 1), lambda i: (i, 0)),
            pl.BlockSpec((1, H1), lambda i: (0, 0)),
            pl.BlockSpec((1, H1), lambda i: (0, 0)),
        ],
        out_shape=[
            jax.ShapeDtypeStruct((B, H1), jnp.float32),
            jax.ShapeDtypeStruct((B, 1), jnp.float32),
            jax.ShapeDtypeStruct((1, H1), jnp.float32),
            jax.ShapeDtypeStruct((1, H1), jnp.float32),
        ],
        scratch_shapes=[
            pltpu.VMEM((1, H1), jnp.float32),
            pltpu.VMEM((1, H1), jnp.float32),
        ],
    )(x, o1v, w0, b0, s1, t1, w1, b1, g2, bt2)


def _k3_body(h1_ref, side_ref, s2_ref, t2_ref, w2_ref, b2_ref, w3_ref, b3_ref,
             out_ref):
    a2 = jnp.maximum(h1_ref[...] * s2_ref[...] + t2_ref[...], 0.0)
    h2 = jnp.dot(a2, w2_ref[...], preferred_element_type=jnp.float32) + b2_ref[...]
    dnn = jnp.dot(h2, w3_ref[...], preferred_element_type=jnp.float32) + b3_ref[...]
    out_ref[...] = dnn + side_ref[...]


def _k3(h1, side, s2, t2, w2, b2, w3, b3):
    return pl.pallas_call(
        _k3_body,
        grid=(_NB,),
        in_specs=[
            pl.BlockSpec((_BM, H1), lambda i: (i, 0)),
            pl.BlockSpec((_BM, 1), lambda i: (i, 0)),
            pl.BlockSpec((1, H1), lambda i: (0, 0)),
            pl.BlockSpec((1, H1), lambda i: (0, 0)),
            pl.BlockSpec((H1, H2), lambda i: (0, 0)),
            pl.BlockSpec((1, H2), lambda i: (0, 0)),
            pl.BlockSpec((H2, 1), lambda i: (0, 0)),
            pl.BlockSpec((1, 1), lambda i: (0, 0)),
        ],
        out_specs=pl.BlockSpec((_BM, 1), lambda i: (i, 0)),
        out_shape=jax.ShapeDtypeStruct((B, 1), jnp.float32),
    )(h1, side, s2, t2, w2, b2, w3, b3)


def kernel(x, cat_embed, o1_fc, W0, b0, g1, beta1, W1, b1, g2, beta2, W2, b2,
           W3, b3):
    xf = x.astype(jnp.int32).reshape(NIDX // _G, _G)
    dims = [cat_embed[:, d].reshape(1, TOTAL) for d in range(D)]
    eT, o1g = _sc_gather(xf, dims, o1_fc.T)
    X = eT.reshape(D, B, F).transpose(1, 2, 0).reshape(B, DIN)
    o1v = o1g.reshape(B, F)
    r = lambda a: a.reshape(1, -1)
    s1, t1 = _k1(X, W0, r(b0), r(g1), r(beta1))
    h1, side, s2, t2 = _k2(X, o1v, W0, r(b0), s1, t1, W1, r(b1), r(g2), r(beta2))
    return _k3(h1, side, s2, t2, W2, r(b2), W3, r(b3))


# speedup vs baseline: 1.0951x; 1.0951x over previous
import functools

import jax
import jax.numpy as jnp
from jax import lax
from jax.experimental import pallas as pl
from jax.experimental.pallas import tpu as pltpu
from jax.experimental.pallas import tpu_sc as plsc

B = 16384
F = 26
D = 16
DIN = F * D
H0, H1, H2 = 1024, 512, 256
EPS = 1e-5
NIDX = B * F
TOTAL = 2600000

_NC, _NS = 2, 16
_NW = _NC * _NS
_PER_W = NIDX // _NW
_G = 128
_GROUPS = _PER_W // _G
_CH_G = 8
_CH = _CH_G * _G
_NCH = _GROUPS // _CH_G


def _sc_gather(xf, emb, o1f):
    @functools.partial(
        pl.kernel,
        out_type=(
            jax.ShapeDtypeStruct((NIDX, D), jnp.float32),
            jax.ShapeDtypeStruct((NIDX,), jnp.float32),
        ),
        mesh=plsc.VectorSubcoreMesh(core_axis_name="c", subcore_axis_name="s"),
        compiler_params=pltpu.CompilerParams(use_tc_tiling_on_sc=False),
        scratch_types=[
            pltpu.VMEM((_CH_G, _G), jnp.int32),
            pltpu.VMEM((_CH, D), jnp.float32),
            pltpu.VMEM((_CH,), jnp.float32),
            pltpu.SemaphoreType.DMA,
        ],
    )
    def k(idx_hbm, emb_hbm, o1_hbm, rows_out, o1_out, idx_v, rows_v, o1_v, sem):
        wid = lax.axis_index("s") * _NC + lax.axis_index("c")

        def chunk(i, carry):
            row0 = wid * _GROUPS + i * _CH_G
            base = wid * _PER_W + i * _CH
            pltpu.sync_copy(idx_hbm.at[pl.ds(row0, _CH_G)], idx_v)
            for g in range(_CH_G):
                pltpu.async_copy(
                    emb_hbm.at[idx_v.at[g]], rows_v.at[pl.ds(g * _G, _G)], sem)
                pltpu.async_copy(
                    o1_hbm.at[0].at[idx_v.at[g]], o1_v.at[pl.ds(g * _G, _G)],
                    sem)
            pltpu.make_async_copy(rows_out.at[pl.ds(0, _CH)], rows_v, sem).wait()
            pltpu.make_async_copy(o1_out.at[pl.ds(0, _CH)], o1_v, sem).wait()
            pltpu.sync_copy(rows_v, rows_out.at[pl.ds(base, _CH)])
            pltpu.sync_copy(o1_v, o1_out.at[pl.ds(base, _CH)])
            return carry

        lax.fori_loop(0, _NCH, chunk, 0)

    return k(xf, emb, o1f)


_BM = 512
_NB = B // _BM


def _k1_body(x_ref, w0_ref, b0_ref, g1_ref, bt1_ref, s1_ref, t1_ref,
             c_acc, sum_acc):
    i = pl.program_id(0)

    @pl.when(i == 0)
    def _():
        c_acc[...] = jnp.zeros_like(c_acc)
        sum_acc[...] = jnp.zeros_like(sum_acc)

    xb = x_ref[...]
    c_acc[...] += lax.dot_general(
        xb, xb, (((0,), (0,)), ((), ())), preferred_element_type=jnp.float32)
    sum_acc[...] += jnp.sum(xb, axis=0, keepdims=True)

    @pl.when(i == _NB - 1)
    def _():
        c = c_acc[...] * (1.0 / B)
        m = sum_acc[...] * (1.0 / B)
        w0 = w0_ref[...]
        b0 = b0_ref[...]
        p = jnp.dot(c, w0, preferred_element_type=jnp.float32)
        quad = jnp.sum(w0 * p, axis=0, keepdims=True)
        mm = jnp.dot(m, w0, preferred_element_type=jnp.float32)
        mean = mm + b0
        ex2 = quad + 2.0 * b0 * mm + b0 * b0
        var = ex2 - mean * mean
        s1 = g1_ref[...] * lax.rsqrt(var + EPS)
        s1_ref[...] = s1
        t1_ref[...] = bt1_ref[...] - mean * s1


def _k1(x, w0, b0, g1, bt1):
    return pl.pallas_call(
        _k1_body,
        grid=(_NB,),
        in_specs=[
            pl.BlockSpec((_BM, DIN), lambda i: (i, 0)),
            pl.BlockSpec((DIN, H0), lambda i: (0, 0)),
            pl.BlockSpec((1, H0), lambda i: (0, 0)),
            pl.BlockSpec((1, H0), lambda i: (0, 0)),
            pl.BlockSpec((1, H0), lambda i: (0, 0)),
        ],
        out_specs=[
            pl.BlockSpec((1, H0), lambda i: (0, 0)),
            pl.BlockSpec((1, H0), lambda i: (0, 0)),
        ],
        out_shape=[
            jax.ShapeDtypeStruct((1, H0), jnp.float32),
            jax.ShapeDtypeStruct((1, H0), jnp.float32),
        ],
        scratch_shapes=[
            pltpu.VMEM((DIN, DIN), jnp.float32),
            pltpu.VMEM((1, DIN), jnp.float32),
        ],
    )(x, w0, b0, g1, bt1)


def _k2_body(x_ref, o1v_ref, w0_ref, b0_ref, s1_ref, t1_ref, w1_ref, b1_ref,
             g2_ref, bt2_ref, h1_ref, side_ref, s2_ref, t2_ref,
             sum_acc, sq_acc):
    i = pl.program_id(0)

    @pl.when(i == 0)
    def _():
        sum_acc[...] = jnp.zeros_like(sum_acc)
        sq_acc[...] = jnp.zeros_like(sq_acc)

    xb = x_ref[...]
    h0 = jnp.dot(xb, w0_ref[...], preferred_element_type=jnp.float32) + b0_ref[...]
    a1 = jnp.maximum(h0 * s1_ref[...] + t1_ref[...], 0.0)
    h1 = jnp.dot(a1, w1_ref[...], preferred_element_type=jnp.float32) + b1_ref[...]
    h1_ref[...] = h1
    sum_acc[...] += jnp.sum(h1, axis=0, keepdims=True)
    sq_acc[...] += jnp.sum(h1 * h1, axis=0, keepdims=True)

    rows = lax.broadcasted_iota(jnp.int32, (DIN, D), 0)
    cols = lax.broadcasted_iota(jnp.int32, (DIN, D), 1)
    sel = jnp.where(rows % D == cols, 1.0, 0.0).astype(jnp.float32)
    se = jnp.dot(xb, sel, preferred_element_type=jnp.float32)
    sq = jnp.dot(xb * xb, sel, preferred_element_type=jnp.float32)
    o2 = 0.5 * jnp.sum(se * se - sq, axis=1, keepdims=True)
    o1 = lax.dot_general(o1v_ref[...], jnp.ones((F, 1), jnp.float32),
                         (((1,), (0,)), ((), ())),
                         preferred_element_type=jnp.float32)
    side_ref[...] = o1 + o2

    @pl.when(i == _NB - 1)
    def _():
        mean = sum_acc[...] * (1.0 / B)
        var = sq_acc[...] * (1.0 / B) - mean * mean
        s2 = g2_ref[...] * lax.rsqrt(var + EPS)
        s2_ref[...] = s2
        t2_ref[...] = bt2_ref[...] - mean * s2


def _k2(x, o1v, w0, b0, s1, t1, w1, b1, g2, bt2):
    return pl.pallas_call(
        _k2_body,
        grid=(_NB,),
        in_specs=[
            pl.BlockSpec((_BM, DIN), lambda i: (i, 0)),
            pl.BlockSpec((_BM, F), lambda i: (i, 0)),
            pl.BlockSpec((DIN, H0), lambda i: (0, 0)),
            pl.BlockSpec((1, H0), lambda i: (0, 0)),
            pl.BlockSpec((1, H0), lambda i: (0, 0)),
            pl.BlockSpec((1, H0), lambda i: (0, 0)),
            pl.BlockSpec((H0, H1), lambda i: (0, 0)),
            pl.BlockSpec((1, H1), lambda i: (0, 0)),
            pl.BlockSpec((1, H1), lambda i: (0, 0)),
            pl.BlockSpec((1, H1), lambda i: (0, 0)),
        ],
        out_specs=[
            pl.BlockSpec((_BM, H1), lambda i: (i, 0)),
            pl.BlockSpec((_BM, 1), lambda i: (i, 0)),
            pl.BlockSpec((1, H1), lambda i: (0, 0)),
            pl.BlockSpec((1, H1), lambda i: (0, 0)),
        ],
        out_shape=[
            jax.ShapeDtypeStruct((B, H1), jnp.float32),
            jax.ShapeDtypeStruct((B, 1), jnp.float32),
            jax.ShapeDtypeStruct((1, H1), jnp.float32),
            jax.ShapeDtypeStruct((1, H1), jnp.float32),
        ],
        scratch_shapes=[
            pltpu.VMEM((1, H1), jnp.float32),
            pltpu.VMEM((1, H1), jnp.float32),
        ],
    )(x, o1v, w0, b0, s1, t1, w1, b1, g2, bt2)


def _k3_body(h1_ref, side_ref, s2_ref, t2_ref, w2_ref, b2_ref, w3_ref, b3_ref,
             out_ref):
    a2 = jnp.maximum(h1_ref[...] * s2_ref[...] + t2_ref[...], 0.0)
    h2 = jnp.dot(a2, w2_ref[...], preferred_element_type=jnp.float32) + b2_ref[...]
    dnn = jnp.dot(h2, w3_ref[...], preferred_element_type=jnp.float32) + b3_ref[...]
    out_ref[...] = dnn + side_ref[...]


def _k3(h1, side, s2, t2, w2, b2, w3, b3):
    return pl.pallas_call(
        _k3_body,
        grid=(_NB,),
        in_specs=[
            pl.BlockSpec((_BM, H1), lambda i: (i, 0)),
            pl.BlockSpec((_BM, 1), lambda i: (i, 0)),
            pl.BlockSpec((1, H1), lambda i: (0, 0)),
            pl.BlockSpec((1, H1), lambda i: (0, 0)),
            pl.BlockSpec((H1, H2), lambda i: (0, 0)),
            pl.BlockSpec((1, H2), lambda i: (0, 0)),
            pl.BlockSpec((H2, 1), lambda i: (0, 0)),
            pl.BlockSpec((1, 1), lambda i: (0, 0)),
        ],
        out_specs=pl.BlockSpec((_BM, 1), lambda i: (i, 0)),
        out_shape=jax.ShapeDtypeStruct((B, 1), jnp.float32),
    )(h1, side, s2, t2, w2, b2, w3, b3)


def kernel(x, cat_embed, o1_fc, W0, b0, g1, beta1, W1, b1, g2, beta2, W2, b2,
           W3, b3):
    xf = x.astype(jnp.int32).reshape(NIDX // _G, _G)
    rows, o1g = _sc_gather(xf, cat_embed, o1_fc.T)
    X = rows.reshape(B, DIN)
    o1v = o1g.reshape(B, F)
    r = lambda a: a.reshape(1, -1)
    s1, t1 = _k1(X, W0, r(b0), r(g1), r(beta1))
    h1, side, s2, t2 = _k2(X, o1v, W0, r(b0), s1, t1, W1, r(b1), r(g2), r(beta2))
    return _k3(h1, side, s2, t2, W2, r(b2), W3, r(b3))
